# baseline (speedup 1.0000x reference)
import functools

import jax
import jax.numpy as jnp
from jax import lax
from jax.experimental import pallas as pl
from jax.experimental.pallas import tpu as pltpu
from jax.experimental.pallas import tpu_sc as plsc

N = 10000
E = 320000
D = 128

NC = 2
NS = 16
NW = NC * NS
EW = E // NW
C = 40
NCH = EW // C
NB = 50
NBLK = NCH // NB
RPT = 624
TAIL0 = NS * RPT
TAILN = N - TAIL0

_f32 = jnp.float32


def _sc_agg_body(with_cnt, *refs):
    (h_hbm, src_hbm, dst_hbm, znd_hbm, zn_hbm, acc_hbm, cnt_hbm,
     sidx, didx, rows0, rows1, rows2, rows3, ones_v, acc, cnt,
     gs0, gs1, gs2, gs3, ss0, ss1, ss2, ss3) = refs
    rows = (rows0, rows1, rows2, rows3)
    gs = (gs0, gs1, gs2, gs3)
    ss = (ss0, ss1, ss2, ss3)
    c = lax.axis_index("c")
    s = lax.axis_index("s")
    w = c * NS + s
    r0 = s * RPT

    pltpu.sync_copy(src_hbm.at[w, 0], sidx)
    pltpu.sync_copy(dst_hbm.at[w, 0], didx)
    pltpu.async_copy(h_hbm.at[sidx.at[0]], rows0, gs0)
    pltpu.async_copy(h_hbm.at[sidx.at[1]], rows1, gs1)
    pltpu.async_copy(h_hbm.at[sidx.at[2]], rows2, gs2)

    pltpu.sync_copy(znd_hbm.at[pl.ds(r0, RPT)], acc.at[pl.ds(r0, RPT)])

    @pl.when(s == NS - 1)
    def _():
        pltpu.sync_copy(znd_hbm.at[pl.ds(TAIL0, TAILN)],
                        acc.at[pl.ds(TAIL0, TAILN)])

    if with_cnt:
        @pl.when(s == 0)
        def _():
            pltpu.sync_copy(zn_hbm, cnt)

        for k in (0, 16, 24):
            ones_v[pl.ds(k, 16)] = jnp.ones((16,), _f32)

    plsc.subcore_barrier()

    def wait_scat(rowbuf, sem):
        pltpu.make_async_copy(h_hbm.at[sidx.at[0]], rowbuf, sem).wait()
        if with_cnt:
            pltpu.make_async_copy(zn_hbm.at[sidx.at[0]], ones_v, sem).wait()

    def chunk(m, buf, first):
        pltpu.make_async_copy(h_hbm.at[sidx.at[m]], rows[buf], gs[buf]).wait()
        pltpu.async_copy(rows[buf], acc.at[didx.at[m]], ss[buf], add=True)
        if with_cnt:
            pltpu.async_copy(ones_v, cnt.at[didx.at[m]], ss[buf], add=True)
        nbuf = (buf + 3) % 4
        if first:
            pltpu.async_copy(h_hbm.at[sidx.at[m + 3]], rows[nbuf], gs[nbuf])
        else:
            wait_scat(rows[nbuf], ss[nbuf])

            @pl.when(m + 3 < NB)
            def _():
                pltpu.async_copy(h_hbm.at[sidx.at[m + 3]], rows[nbuf],
                                 gs[nbuf])

    def run_block():
        chunk(0, 0, True)

        def quad(i, carry2):
            m = 4 * i + 1
            chunk(m + 0, 1, False)
            chunk(m + 1, 2, False)
            chunk(m + 2, 3, False)
            chunk(m + 3, 0, False)
            return carry2

        lax.fori_loop(0, (NB - 2) // 4, quad, 0)
        chunk(NB - 1, (NB - 1) % 4, False)
        wait_scat(rows[(NB - 1) % 4], ss[(NB - 1) % 4])

    run_block()

    def block(b, carry):
        pltpu.sync_copy(src_hbm.at[w, b], sidx)
        pltpu.sync_copy(dst_hbm.at[w, b], didx)
        pltpu.async_copy(h_hbm.at[sidx.at[0]], rows0, gs0)
        pltpu.async_copy(h_hbm.at[sidx.at[1]], rows1, gs1)
        pltpu.async_copy(h_hbm.at[sidx.at[2]], rows2, gs2)
        run_block()
        return carry

    lax.fori_loop(1, NBLK, block, 0)

    plsc.subcore_barrier()

    pltpu.sync_copy(acc.at[pl.ds(r0, RPT)], acc_hbm.at[c, pl.ds(r0, RPT)])

    @pl.when(s == NS - 1)
    def _():
        pltpu.sync_copy(acc.at[pl.ds(TAIL0, TAILN)],
                        acc_hbm.at[c, pl.ds(TAIL0, TAILN)])

    if with_cnt:
        @pl.when(s == 0)
        def _():
            pltpu.sync_copy(cnt, cnt_hbm.at[c])


_sc_mesh = plsc.VectorSubcoreMesh(
    core_axis_name="c", subcore_axis_name="s",
    num_cores=NC, num_subcores=NS)

_sc_agg_cnt = pl.kernel(
    functools.partial(_sc_agg_body, True),
    out_type=[
        jax.ShapeDtypeStruct((NC, N, D), _f32),
        jax.ShapeDtypeStruct((NC, N), _f32),
    ],
    mesh=_sc_mesh,
    scratch_types=[
        pltpu.VMEM((NB, C), jnp.int32),
        pltpu.VMEM((NB, C), jnp.int32),
        pltpu.VMEM((C, D), _f32),
        pltpu.VMEM((C, D), _f32),
        pltpu.VMEM((C, D), _f32),
        pltpu.VMEM((C, D), _f32),
        pltpu.VMEM((C,), _f32),
        pltpu.VMEM_SHARED((N, D), _f32),
        pltpu.VMEM_SHARED((N,), _f32),
        pltpu.SemaphoreType.DMA,
        pltpu.SemaphoreType.DMA,
        pltpu.SemaphoreType.DMA,
        pltpu.SemaphoreType.DMA,
        pltpu.SemaphoreType.DMA,
        pltpu.SemaphoreType.DMA,
        pltpu.SemaphoreType.DMA,
        pltpu.SemaphoreType.DMA,
    ],
)


BN = 1000


def _combine_body(relu, acc_ref, cnt_ref, x_ref, wl_ref, bl_ref, wr_ref, o_ref):
    cnt = cnt_ref[:, 0] + cnt_ref[:, 1]
    denom = jnp.maximum(cnt, 1.0)
    agg = acc_ref[0] + acc_ref[1]
    mean = agg / denom[:, None]
    h = jnp.dot(mean, wl_ref[...], preferred_element_type=_f32)
    h = h + jnp.dot(x_ref[...], wr_ref[...], preferred_element_type=_f32)
    h = h + bl_ref[...]
    o_ref[...] = jnp.maximum(h, 0.0) if relu else h


def _make_combine(relu):
    return pl.pallas_call(
        functools.partial(_combine_body, relu),
        grid=(N // BN,),
        in_specs=[
            pl.BlockSpec((NC, BN, D), lambda i: (0, i, 0)),
            pl.BlockSpec((BN, NC), lambda i: (i, 0)),
            pl.BlockSpec((BN, D), lambda i: (i, 0)),
            pl.BlockSpec((D, D), lambda i: (0, 0)),
            pl.BlockSpec((1, D), lambda i: (0, 0)),
            pl.BlockSpec((D, D), lambda i: (0, 0)),
        ],
        out_specs=pl.BlockSpec((BN, D), lambda i: (i, 0)),
        out_shape=jax.ShapeDtypeStruct((N, D), _f32),
    )


_combine_relu = _make_combine(True)


def _final_body(acc_ref, cnt_ref, x_ref, wl_ref, bl_ref, wr_ref,
                w1_ref, b1_ref, w2_ref, b2_ref, lsm_ref, h3_ref):
    cnt = cnt_ref[:, 0] + cnt_ref[:, 1]
    denom = jnp.maximum(cnt, 1.0)
    agg = acc_ref[0] + acc_ref[1]
    mean = agg / denom[:, None]
    h3 = jnp.dot(mean, wl_ref[...], preferred_element_type=_f32)
    h3 = h3 + jnp.dot(x_ref[...], wr_ref[...], preferred_element_type=_f32)
    h3 = h3 + bl_ref[...]
    z = jnp.maximum(jnp.dot(h3, w1_ref[...], preferred_element_type=_f32)
                    + b1_ref[...], 0.0)
    z = jnp.dot(z, w2_ref[...], preferred_element_type=_f32) + b2_ref[...]
    m = jnp.max(z, axis=1, keepdims=True)
    lse = m + jnp.log(jnp.sum(jnp.exp(z - m), axis=1, keepdims=True))
    lsm_ref[...] = z - lse
    h3_ref[...] = h3


_final = pl.pallas_call(
    _final_body,
    grid=(N // BN,),
    in_specs=[
        pl.BlockSpec((NC, BN, D), lambda i: (0, i, 0)),
        pl.BlockSpec((BN, NC), lambda i: (i, 0)),
        pl.BlockSpec((BN, D), lambda i: (i, 0)),
        pl.BlockSpec((D, D), lambda i: (0, 0)),
        pl.BlockSpec((1, D), lambda i: (0, 0)),
        pl.BlockSpec((D, D), lambda i: (0, 0)),
        pl.BlockSpec((D, D), lambda i: (0, 0)),
        pl.BlockSpec((1, D), lambda i: (0, 0)),
        pl.BlockSpec((D, D), lambda i: (0, 0)),
        pl.BlockSpec((1, D), lambda i: (0, 0)),
    ],
    out_specs=[
        pl.BlockSpec((BN, D), lambda i: (i, 0)),
        pl.BlockSpec((BN, D), lambda i: (i, 0)),
    ],
    out_shape=[
        jax.ShapeDtypeStruct((N, D), _f32),
        jax.ShapeDtypeStruct((N, D), _f32),
    ],
)


def kernel(x, edge_index, Wl1, bl1, Wr1, Wl2, bl2, Wr2, Wl3, bl3, Wr3,
           W_fc1, b_fc1, W_fc2, b_fc2):
    src2 = edge_index[0].reshape(NW, NBLK, NB, C)
    dst2 = edge_index[1].reshape(NW, NBLK, NB, C)
    znd = jnp.zeros((N, D), _f32)
    zn = jnp.zeros((N,), _f32)

    agg1, cnt2 = _sc_agg_cnt(x, src2, dst2, znd, zn)
    cntT = cnt2.T
    h1 = _combine_relu(agg1, cntT, x, Wl1.T, bl1.reshape(1, D), Wr1.T)
    agg2, _ = _sc_agg_cnt(h1, src2, dst2, znd, zn)
    h2 = _combine_relu(agg2, cntT, h1, Wl2.T, bl2.reshape(1, D), Wr2.T)
    agg3, _ = _sc_agg_cnt(h2, src2, dst2, znd, zn)
    lsm, h3 = _final(agg3, cntT, h2, Wl3.T, bl3.reshape(1, D), Wr3.T,
                     W_fc1.T, b_fc1.reshape(1, D), W_fc2.T, b_fc2.reshape(1, D))
    return (lsm, h3)


# speedup vs baseline: 1.0409x; 1.0409x over previous
import functools

import jax
import jax.numpy as jnp
from jax import lax
from jax.experimental import pallas as pl
from jax.experimental.pallas import tpu as pltpu
from jax.experimental.pallas import tpu_sc as plsc

N = 10000
E = 320000
D = 128

NC = 2
NS = 16
NW = NC * NS
EW = E // NW
C = 80
NCH = EW // C
NB = 25
NBLK = NCH // NB
RPT = 624
TAIL0 = NS * RPT
TAILN = N - TAIL0

_f32 = jnp.float32


def _sc_agg_body(with_cnt, *refs):
    (h_hbm, src_hbm, dst_hbm, znd_hbm, zn_hbm, acc_hbm, cnt_hbm,
     sidx, didx, rows0, rows1, rows2, rows3, ones_v, acc, cnt,
     gs0, gs1, gs2, gs3, ss0, ss1, ss2, ss3) = refs
    rows = (rows0, rows1, rows2, rows3)
    gs = (gs0, gs1, gs2, gs3)
    ss = (ss0, ss1, ss2, ss3)
    c = lax.axis_index("c")
    s = lax.axis_index("s")
    w = c * NS + s
    r0 = s * RPT

    pltpu.sync_copy(src_hbm.at[w, 0], sidx)
    pltpu.sync_copy(dst_hbm.at[w, 0], didx)
    pltpu.async_copy(h_hbm.at[sidx.at[0]], rows0, gs0)
    pltpu.async_copy(h_hbm.at[sidx.at[1]], rows1, gs1)
    pltpu.async_copy(h_hbm.at[sidx.at[2]], rows2, gs2)

    pltpu.sync_copy(znd_hbm.at[pl.ds(r0, RPT)], acc.at[pl.ds(r0, RPT)])

    @pl.when(s == NS - 1)
    def _():
        pltpu.sync_copy(znd_hbm.at[pl.ds(TAIL0, TAILN)],
                        acc.at[pl.ds(TAIL0, TAILN)])

    if with_cnt:
        @pl.when(s == 0)
        def _():
            pltpu.sync_copy(zn_hbm, cnt)

        for k in range(C // 16):
            ones_v[pl.ds(k * 16, 16)] = jnp.ones((16,), _f32)

    plsc.subcore_barrier()

    def wait_scat(rowbuf, sem):
        pltpu.make_async_copy(h_hbm.at[sidx.at[0]], rowbuf, sem).wait()
        if with_cnt:
            pltpu.make_async_copy(zn_hbm.at[sidx.at[0]], ones_v, sem).wait()

    def chunk(m, buf, first):
        pltpu.make_async_copy(h_hbm.at[sidx.at[m]], rows[buf], gs[buf]).wait()
        pltpu.async_copy(rows[buf], acc.at[didx.at[m]], ss[buf], add=True)
        if with_cnt:
            pltpu.async_copy(ones_v, cnt.at[didx.at[m]], ss[buf], add=True)
        nbuf = (buf + 3) % 4
        if first:
            pltpu.async_copy(h_hbm.at[sidx.at[m + 3]], rows[nbuf], gs[nbuf])
        else:
            wait_scat(rows[nbuf], ss[nbuf])

            @pl.when(m + 3 < NB)
            def _():
                pltpu.async_copy(h_hbm.at[sidx.at[m + 3]], rows[nbuf],
                                 gs[nbuf])

    def run_block():
        chunk(0, 0, True)

        def quad(i, carry2):
            m = 4 * i + 1
            chunk(m + 0, 1, False)
            chunk(m + 1, 2, False)
            chunk(m + 2, 3, False)
            chunk(m + 3, 0, False)
            return carry2

        lax.fori_loop(0, (NB - 1) // 4, quad, 0)
        wait_scat(rows[(NB - 1) % 4], ss[(NB - 1) % 4])

    run_block()

    def block(b, carry):
        pltpu.sync_copy(src_hbm.at[w, b], sidx)
        pltpu.sync_copy(dst_hbm.at[w, b], didx)
        pltpu.async_copy(h_hbm.at[sidx.at[0]], rows0, gs0)
        pltpu.async_copy(h_hbm.at[sidx.at[1]], rows1, gs1)
        pltpu.async_copy(h_hbm.at[sidx.at[2]], rows2, gs2)
        run_block()
        return carry

    lax.fori_loop(1, NBLK, block, 0)

    plsc.subcore_barrier()

    pltpu.sync_copy(acc.at[pl.ds(r0, RPT)], acc_hbm.at[c, pl.ds(r0, RPT)])

    @pl.when(s == NS - 1)
    def _():
        pltpu.sync_copy(acc.at[pl.ds(TAIL0, TAILN)],
                        acc_hbm.at[c, pl.ds(TAIL0, TAILN)])

    if with_cnt:
        @pl.when(s == 0)
        def _():
            pltpu.sync_copy(cnt, cnt_hbm.at[c])


_sc_mesh = plsc.VectorSubcoreMesh(
    core_axis_name="c", subcore_axis_name="s",
    num_cores=NC, num_subcores=NS)

_sc_agg_cnt = pl.kernel(
    functools.partial(_sc_agg_body, True),
    out_type=[
        jax.ShapeDtypeStruct((NC, N, D), _f32),
        jax.ShapeDtypeStruct((NC, N), _f32),
    ],
    mesh=_sc_mesh,
    scratch_types=[
        pltpu.VMEM((NB, C), jnp.int32),
        pltpu.VMEM((NB, C), jnp.int32),
        pltpu.VMEM((C, D), _f32),
        pltpu.VMEM((C, D), _f32),
        pltpu.VMEM((C, D), _f32),
        pltpu.VMEM((C, D), _f32),
        pltpu.VMEM((C,), _f32),
        pltpu.VMEM_SHARED((N, D), _f32),
        pltpu.VMEM_SHARED((N,), _f32),
        pltpu.SemaphoreType.DMA,
        pltpu.SemaphoreType.DMA,
        pltpu.SemaphoreType.DMA,
        pltpu.SemaphoreType.DMA,
        pltpu.SemaphoreType.DMA,
        pltpu.SemaphoreType.DMA,
        pltpu.SemaphoreType.DMA,
        pltpu.SemaphoreType.DMA,
    ],
)


BN = 1000


def _combine_body(relu, acc_ref, cnt_ref, x_ref, wl_ref, bl_ref, wr_ref, o_ref):
    cnt = cnt_ref[:, 0] + cnt_ref[:, 1]
    denom = jnp.maximum(cnt, 1.0)
    agg = acc_ref[0] + acc_ref[1]
    mean = agg / denom[:, None]
    h = jnp.dot(mean, wl_ref[...], preferred_element_type=_f32)
    h = h + jnp.dot(x_ref[...], wr_ref[...], preferred_element_type=_f32)
    h = h + bl_ref[...]
    o_ref[...] = jnp.maximum(h, 0.0) if relu else h


def _make_combine(relu):
    return pl.pallas_call(
        functools.partial(_combine_body, relu),
        grid=(N // BN,),
        in_specs=[
            pl.BlockSpec((NC, BN, D), lambda i: (0, i, 0)),
            pl.BlockSpec((BN, NC), lambda i: (i, 0)),
            pl.BlockSpec((BN, D), lambda i: (i, 0)),
            pl.BlockSpec((D, D), lambda i: (0, 0)),
            pl.BlockSpec((1, D), lambda i: (0, 0)),
            pl.BlockSpec((D, D), lambda i: (0, 0)),
        ],
        out_specs=pl.BlockSpec((BN, D), lambda i: (i, 0)),
        out_shape=jax.ShapeDtypeStruct((N, D), _f32),
    )


_combine_relu = _make_combine(True)


def _final_body(acc_ref, cnt_ref, x_ref, wl_ref, bl_ref, wr_ref,
                w1_ref, b1_ref, w2_ref, b2_ref, lsm_ref, h3_ref):
    cnt = cnt_ref[:, 0] + cnt_ref[:, 1]
    denom = jnp.maximum(cnt, 1.0)
    agg = acc_ref[0] + acc_ref[1]
    mean = agg / denom[:, None]
    h3 = jnp.dot(mean, wl_ref[...], preferred_element_type=_f32)
    h3 = h3 + jnp.dot(x_ref[...], wr_ref[...], preferred_element_type=_f32)
    h3 = h3 + bl_ref[...]
    z = jnp.maximum(jnp.dot(h3, w1_ref[...], preferred_element_type=_f32)
                    + b1_ref[...], 0.0)
    z = jnp.dot(z, w2_ref[...], preferred_element_type=_f32) + b2_ref[...]
    m = jnp.max(z, axis=1, keepdims=True)
    lse = m + jnp.log(jnp.sum(jnp.exp(z - m), axis=1, keepdims=True))
    lsm_ref[...] = z - lse
    h3_ref[...] = h3


_final = pl.pallas_call(
    _final_body,
    grid=(N // BN,),
    in_specs=[
        pl.BlockSpec((NC, BN, D), lambda i: (0, i, 0)),
        pl.BlockSpec((BN, NC), lambda i: (i, 0)),
        pl.BlockSpec((BN, D), lambda i: (i, 0)),
        pl.BlockSpec((D, D), lambda i: (0, 0)),
        pl.BlockSpec((1, D), lambda i: (0, 0)),
        pl.BlockSpec((D, D), lambda i: (0, 0)),
        pl.BlockSpec((D, D), lambda i: (0, 0)),
        pl.BlockSpec((1, D), lambda i: (0, 0)),
        pl.BlockSpec((D, D), lambda i: (0, 0)),
        pl.BlockSpec((1, D), lambda i: (0, 0)),
    ],
    out_specs=[
        pl.BlockSpec((BN, D), lambda i: (i, 0)),
        pl.BlockSpec((BN, D), lambda i: (i, 0)),
    ],
    out_shape=[
        jax.ShapeDtypeStruct((N, D), _f32),
        jax.ShapeDtypeStruct((N, D), _f32),
    ],
)


def kernel(x, edge_index, Wl1, bl1, Wr1, Wl2, bl2, Wr2, Wl3, bl3, Wr3,
           W_fc1, b_fc1, W_fc2, b_fc2):
    src2 = edge_index[0].reshape(NW, NBLK, NB, C)
    dst2 = edge_index[1].reshape(NW, NBLK, NB, C)
    znd = jnp.zeros((N, D), _f32)
    zn = jnp.zeros((N,), _f32)

    agg1, cnt2 = _sc_agg_cnt(x, src2, dst2, znd, zn)
    cntT = cnt2.T
    h1 = _combine_relu(agg1, cntT, x, Wl1.T, bl1.reshape(1, D), Wr1.T)
    agg2, _ = _sc_agg_cnt(h1, src2, dst2, znd, zn)
    h2 = _combine_relu(agg2, cntT, h1, Wl2.T, bl2.reshape(1, D), Wr2.T)
    agg3, _ = _sc_agg_cnt(h2, src2, dst2, znd, zn)
    lsm, h3 = _final(agg3, cntT, h2, Wl3.T, bl3.reshape(1, D), Wr3.T,
                     W_fc1.T, b_fc1.reshape(1, D), W_fc2.T, b_fc2.reshape(1, D))
    return (lsm, h3)


# speedup vs baseline: 1.0771x; 1.0348x over previous
import functools

import jax
import jax.numpy as jnp
from jax import lax
from jax.experimental import pallas as pl
from jax.experimental.pallas import tpu as pltpu
from jax.experimental.pallas import tpu_sc as plsc

N = 10000
E = 320000
D = 128

NC = 2
NS = 16
NW = NC * NS
EW = E // NW
C = 80
NCH = EW // C
NB = 25
NBLK = NCH // NB
RPT = 624
TAIL0 = NS * RPT
TAILN = N - TAIL0

_f32 = jnp.float32


def _sc_agg_body(with_cnt, *refs):
    (h_hbm, src_hbm, dst_hbm, znd_hbm, zn_hbm, acc_hbm, cnt_hbm,
     sidx, didx, rows0, rows1, rows2, ones_v, acc, cnt,
     gs0, gs1, gs2, ss0, ss1, ss2) = refs
    rows = (rows0, rows1, rows2)
    gs = (gs0, gs1, gs2)
    ss = (ss0, ss1, ss2)
    c = lax.axis_index("c")
    s = lax.axis_index("s")
    w = c * NS + s
    r0 = s * RPT

    pltpu.sync_copy(src_hbm.at[w, 0], sidx)
    pltpu.sync_copy(dst_hbm.at[w, 0], didx)
    pltpu.async_copy(h_hbm.at[sidx.at[0]], rows0, gs0)
    pltpu.async_copy(h_hbm.at[sidx.at[1]], rows1, gs1)

    pltpu.sync_copy(znd_hbm.at[pl.ds(r0, RPT)], acc.at[pl.ds(r0, RPT)])

    @pl.when(s == NS - 1)
    def _():
        pltpu.sync_copy(znd_hbm.at[pl.ds(TAIL0, TAILN)],
                        acc.at[pl.ds(TAIL0, TAILN)])

    if with_cnt:
        @pl.when(s == 0)
        def _():
            pltpu.sync_copy(zn_hbm, cnt)

        for k in range(C // 16):
            ones_v[pl.ds(k * 16, 16)] = jnp.ones((16,), _f32)

    plsc.subcore_barrier()

    def wait_scat(rowbuf, sem):
        pltpu.make_async_copy(h_hbm.at[sidx.at[0]], rowbuf, sem).wait()
        if with_cnt:
            pltpu.make_async_copy(zn_hbm.at[sidx.at[0]], ones_v, sem).wait()

    def chunk(m, buf, first):
        pltpu.make_async_copy(h_hbm.at[sidx.at[m]], rows[buf], gs[buf]).wait()
        pltpu.async_copy(rows[buf], acc.at[didx.at[m]], ss[buf], add=True)
        if with_cnt:
            pltpu.async_copy(ones_v, cnt.at[didx.at[m]], ss[buf], add=True)
        nbuf = (buf + 2) % 3
        if first:
            pltpu.async_copy(h_hbm.at[sidx.at[m + 2]], rows[nbuf], gs[nbuf])
        else:
            wait_scat(rows[nbuf], ss[nbuf])

            @pl.when(m + 2 < NB)
            def _():
                pltpu.async_copy(h_hbm.at[sidx.at[m + 2]], rows[nbuf],
                                 gs[nbuf])

    def run_block():
        chunk(0, 0, True)
        chunk(1, 1, False)

        def triple(i, carry2):
            m = 3 * i + 2
            chunk(m + 0, 2, False)
            chunk(m + 1, 0, False)
            chunk(m + 2, 1, False)
            return carry2

        lax.fori_loop(0, (NB - 4) // 3, triple, 0)
        chunk(NB - 2, (NB - 2) % 3, False)
        chunk(NB - 1, (NB - 1) % 3, False)
        wait_scat(rows[(NB - 1) % 3], ss[(NB - 1) % 3])

    run_block()

    def block(b, carry):
        pltpu.sync_copy(src_hbm.at[w, b], sidx)
        pltpu.sync_copy(dst_hbm.at[w, b], didx)
        pltpu.async_copy(h_hbm.at[sidx.at[0]], rows0, gs0)
        pltpu.async_copy(h_hbm.at[sidx.at[1]], rows1, gs1)
        run_block()
        return carry

    lax.fori_loop(1, NBLK, block, 0)

    plsc.subcore_barrier()

    pltpu.sync_copy(acc.at[pl.ds(r0, RPT)], acc_hbm.at[c, pl.ds(r0, RPT)])

    @pl.when(s == NS - 1)
    def _():
        pltpu.sync_copy(acc.at[pl.ds(TAIL0, TAILN)],
                        acc_hbm.at[c, pl.ds(TAIL0, TAILN)])

    if with_cnt:
        @pl.when(s == 0)
        def _():
            pltpu.sync_copy(cnt, cnt_hbm.at[c])


_sc_mesh = plsc.VectorSubcoreMesh(
    core_axis_name="c", subcore_axis_name="s",
    num_cores=NC, num_subcores=NS)

_sc_agg_cnt = pl.kernel(
    functools.partial(_sc_agg_body, True),
    out_type=[
        jax.ShapeDtypeStruct((NC, N, D), _f32),
        jax.ShapeDtypeStruct((NC, N), _f32),
    ],
    mesh=_sc_mesh,
    scratch_types=[
        pltpu.VMEM((NB, C), jnp.int32),
        pltpu.VMEM((NB, C), jnp.int32),
        pltpu.VMEM((C, D), _f32),
        pltpu.VMEM((C, D), _f32),
        pltpu.VMEM((C, D), _f32),
        pltpu.VMEM((C,), _f32),
        pltpu.VMEM_SHARED((N, D), _f32),
        pltpu.VMEM_SHARED((N,), _f32),
        pltpu.SemaphoreType.DMA,
        pltpu.SemaphoreType.DMA,
        pltpu.SemaphoreType.DMA,
        pltpu.SemaphoreType.DMA,
        pltpu.SemaphoreType.DMA,
        pltpu.SemaphoreType.DMA,
    ],
)


BN = 2000


def _combine_body(relu, acc_ref, cnt_ref, x_ref, wl_ref, bl_ref, wr_ref, o_ref):
    cnt = cnt_ref[:, 0] + cnt_ref[:, 1]
    denom = jnp.maximum(cnt, 1.0)
    agg = acc_ref[0] + acc_ref[1]
    mean = agg / denom[:, None]
    h = jnp.dot(mean, wl_ref[...], preferred_element_type=_f32)
    h = h + jnp.dot(x_ref[...], wr_ref[...], preferred_element_type=_f32)
    h = h + bl_ref[...]
    o_ref[...] = jnp.maximum(h, 0.0) if relu else h


def _make_combine(relu):
    return pl.pallas_call(
        functools.partial(_combine_body, relu),
        grid=(N // BN,),
        in_specs=[
            pl.BlockSpec((NC, BN, D), lambda i: (0, i, 0)),
            pl.BlockSpec((BN, NC), lambda i: (i, 0)),
            pl.BlockSpec((BN, D), lambda i: (i, 0)),
            pl.BlockSpec((D, D), lambda i: (0, 0)),
            pl.BlockSpec((1, D), lambda i: (0, 0)),
            pl.BlockSpec((D, D), lambda i: (0, 0)),
        ],
        out_specs=pl.BlockSpec((BN, D), lambda i: (i, 0)),
        out_shape=jax.ShapeDtypeStruct((N, D), _f32),
    )


_combine_relu = _make_combine(True)


def _final_body(acc_ref, cnt_ref, x_ref, wl_ref, bl_ref, wr_ref,
                w1_ref, b1_ref, w2_ref, b2_ref, lsm_ref, h3_ref):
    cnt = cnt_ref[:, 0] + cnt_ref[:, 1]
    denom = jnp.maximum(cnt, 1.0)
    agg = acc_ref[0] + acc_ref[1]
    mean = agg / denom[:, None]
    h3 = jnp.dot(mean, wl_ref[...], preferred_element_type=_f32)
    h3 = h3 + jnp.dot(x_ref[...], wr_ref[...], preferred_element_type=_f32)
    h3 = h3 + bl_ref[...]
    z = jnp.maximum(jnp.dot(h3, w1_ref[...], preferred_element_type=_f32)
                    + b1_ref[...], 0.0)
    z = jnp.dot(z, w2_ref[...], preferred_element_type=_f32) + b2_ref[...]
    m = jnp.max(z, axis=1, keepdims=True)
    lse = m + jnp.log(jnp.sum(jnp.exp(z - m), axis=1, keepdims=True))
    lsm_ref[...] = z - lse
    h3_ref[...] = h3


_final = pl.pallas_call(
    _final_body,
    grid=(N // BN,),
    in_specs=[
        pl.BlockSpec((NC, BN, D), lambda i: (0, i, 0)),
        pl.BlockSpec((BN, NC), lambda i: (i, 0)),
        pl.BlockSpec((BN, D), lambda i: (i, 0)),
        pl.BlockSpec((D, D), lambda i: (0, 0)),
        pl.BlockSpec((1, D), lambda i: (0, 0)),
        pl.BlockSpec((D, D), lambda i: (0, 0)),
        pl.BlockSpec((D, D), lambda i: (0, 0)),
        pl.BlockSpec((1, D), lambda i: (0, 0)),
        pl.BlockSpec((D, D), lambda i: (0, 0)),
        pl.BlockSpec((1, D), lambda i: (0, 0)),
    ],
    out_specs=[
        pl.BlockSpec((BN, D), lambda i: (i, 0)),
        pl.BlockSpec((BN, D), lambda i: (i, 0)),
    ],
    out_shape=[
        jax.ShapeDtypeStruct((N, D), _f32),
        jax.ShapeDtypeStruct((N, D), _f32),
    ],
)


def kernel(x, edge_index, Wl1, bl1, Wr1, Wl2, bl2, Wr2, Wl3, bl3, Wr3,
           W_fc1, b_fc1, W_fc2, b_fc2):
    src2 = edge_index[0].reshape(NW, NBLK, NB, C)
    dst2 = edge_index[1].reshape(NW, NBLK, NB, C)
    znd = jnp.zeros((N, D), _f32)
    zn = jnp.zeros((N,), _f32)

    agg1, cnt2 = _sc_agg_cnt(x, src2, dst2, znd, zn)
    cntT = cnt2.T
    h1 = _combine_relu(agg1, cntT, x, Wl1.T, bl1.reshape(1, D), Wr1.T)
    agg2, _ = _sc_agg_cnt(h1, src2, dst2, znd, zn)
    h2 = _combine_relu(agg2, cntT, h1, Wl2.T, bl2.reshape(1, D), Wr2.T)
    agg3, _ = _sc_agg_cnt(h2, src2, dst2, znd, zn)
    lsm, h3 = _final(agg3, cntT, h2, Wl3.T, bl3.reshape(1, D), Wr3.T,
                     W_fc1.T, b_fc1.reshape(1, D), W_fc2.T, b_fc2.reshape(1, D))
    return (lsm, h3)
